# baseline (speedup 1.0000x reference)
import functools

import jax
import jax.numpy as jnp
from jax import lax
from jax.experimental import pallas as pl
from jax.experimental.pallas import tpu as pltpu
from jax.experimental.pallas import tpu_sc as plsc

HIDDEN = 64
NC, NS = 2, 16
NW = NC * NS
K = 8


@functools.partial(jax.jit, static_argnums=(2, 3))
def _embed_one(idx_flat, table, b_dim, l_dim):
    b_per_w = b_dim // NW
    nb = b_per_w // K

    mesh = plsc.VectorSubcoreMesh(core_axis_name="c", subcore_axis_name="s")

    @functools.partial(
        pl.kernel,
        mesh=mesh,
        out_type=jax.ShapeDtypeStruct((b_dim, l_dim, HIDDEN), jnp.float32),
        scratch_types=[
            [pltpu.VMEM((l_dim,), jnp.int32) for _ in range(K)],
            [pltpu.VMEM((l_dim, HIDDEN), jnp.float32) for _ in range(K)],
            pltpu.SemaphoreType.DMA,
            pltpu.SemaphoreType.DMA,
            pltpu.SemaphoreType.DMA,
        ],
        compiler_params=pltpu.CompilerParams(use_tc_tiling_on_sc=False),
    )
    def emb(idx_hbm, tab_hbm, out_hbm, idx_slots, row_slots,
            isem, gsem, osem):
        wid = lax.axis_index("s") * NC + lax.axis_index("c")
        b0 = pl.multiple_of(wid * b_per_w, b_per_w)


        def fire_idx(t, j):
            off = pl.multiple_of((b0 + t) * l_dim, 8)
            pltpu.async_copy(idx_hbm.at[pl.ds(off, l_dim)], idx_slots[j], isem)

        def wait_idx(t, j):
            off = pl.multiple_of((b0 + t) * l_dim, 8)
            pltpu.make_async_copy(
                idx_hbm.at[pl.ds(off, l_dim)], idx_slots[j], isem).wait()

        def fire_gather(t, j):
            pltpu.async_copy(tab_hbm.at[idx_slots[j]], row_slots[j], gsem)

        def wait_gather(t, j):
            pltpu.make_async_copy(
                tab_hbm.at[idx_slots[j]], row_slots[j], gsem).wait()

        def fire_out(t, j):
            pltpu.async_copy(row_slots[j], out_hbm.at[b0 + t], osem)

        def wait_out(t, j):
            pltpu.make_async_copy(
                row_slots[j], out_hbm.at[b0 + t], osem).wait()

        for j in range(K):
            fire_idx(j, j)
        for j in range(K):
            wait_idx(j, j)
            fire_gather(j, j)

        def batch(g, carry):
            o0 = g * K
            for j in range(K):
                wait_gather(o0 + j, j)
                fire_out(o0 + j, j)
            for j in range(K):
                wait_out(o0 + j, j)
                fire_idx(o0 + K + j, j)
            for j in range(K):
                wait_idx(o0 + K + j, j)
                fire_gather(o0 + K + j, j)
            return carry

        if nb > 1:
            lax.fori_loop(0, nb - 1, batch, 0)
        o0 = (nb - 1) * K
        for j in range(K):
            wait_gather(o0 + j, j)
            fire_out(o0 + j, j)
        for j in range(K):
            wait_out(o0 + j, j)

    return emb(idx_flat, table)


def kernel(speaker_id, phone, train, speaker_id_table, phone_table):
    del train
    b_dim, l_dim = speaker_id.shape
    out_s = _embed_one(speaker_id.reshape(-1), speaker_id_table, b_dim, l_dim)
    out_p = _embed_one(phone.reshape(-1), phone_table, b_dim, l_dim)
    return (out_s, out_p)


# speedup vs baseline: 1.0329x; 1.0329x over previous
import functools

import jax
import jax.numpy as jnp
from jax import lax
from jax.experimental import pallas as pl
from jax.experimental.pallas import tpu as pltpu
from jax.experimental.pallas import tpu_sc as plsc

HIDDEN = 64
NC, NS = 2, 16
NW = NC * NS
K = 8


@functools.partial(jax.jit, static_argnums=(2, 3))
def _embed_one(idx_t, table, b_dim, l_dim):
    b_per_w = b_dim // NW
    nb = l_dim // K

    mesh = plsc.VectorSubcoreMesh(core_axis_name="c", subcore_axis_name="s")

    @functools.partial(
        pl.kernel,
        mesh=mesh,
        out_type=jax.ShapeDtypeStruct((l_dim, b_dim, HIDDEN), jnp.float32),
        scratch_types=[
            [pltpu.VMEM((b_per_w,), jnp.int32) for _ in range(K)],
            [pltpu.VMEM((b_per_w, HIDDEN), jnp.float32) for _ in range(K)],
            pltpu.SemaphoreType.DMA,
            pltpu.SemaphoreType.DMA,
            pltpu.SemaphoreType.DMA,
        ],
        compiler_params=pltpu.CompilerParams(use_tc_tiling_on_sc=False),
    )
    def emb(idx_hbm, tab_hbm, out_hbm, idx_slots, row_slots,
            isem, gsem, osem):
        wid = lax.axis_index("s") * NC + lax.axis_index("c")
        b0 = pl.multiple_of(wid * b_per_w, b_per_w)


        def fire_idx(t, j):
            pltpu.async_copy(
                idx_hbm.at[t, pl.ds(b0, b_per_w)], idx_slots[j], isem)

        def wait_idx(t, j):
            pltpu.make_async_copy(
                idx_hbm.at[t, pl.ds(b0, b_per_w)], idx_slots[j], isem).wait()

        def fire_gather(t, j):
            pltpu.async_copy(tab_hbm.at[idx_slots[j]], row_slots[j], gsem)

        def wait_gather(t, j):
            pltpu.make_async_copy(
                tab_hbm.at[idx_slots[j]], row_slots[j], gsem).wait()

        def fire_out(t, j):
            pltpu.async_copy(
                row_slots[j], out_hbm.at[t, pl.ds(b0, b_per_w)], osem)

        def wait_out(t, j):
            pltpu.make_async_copy(
                row_slots[j], out_hbm.at[t, pl.ds(b0, b_per_w)], osem).wait()

        for j in range(K):
            fire_idx(j, j)
        for j in range(K):
            wait_idx(j, j)
            fire_gather(j, j)

        def batch(g, carry):
            o0 = g * K
            for j in range(K):
                wait_gather(o0 + j, j)
                fire_out(o0 + j, j)
            for j in range(K):
                wait_out(o0 + j, j)
                fire_idx(o0 + K + j, j)
            for j in range(K):
                wait_idx(o0 + K + j, j)
                fire_gather(o0 + K + j, j)
            return carry

        if nb > 1:
            lax.fori_loop(0, nb - 1, batch, 0)
        o0 = (nb - 1) * K
        for j in range(K):
            wait_gather(o0 + j, j)
            fire_out(o0 + j, j)
        for j in range(K):
            wait_out(o0 + j, j)

    return emb(idx_t, table)


def kernel(speaker_id, phone, train, speaker_id_table, phone_table):
    del train
    b_dim, l_dim = speaker_id.shape
    out_s = _embed_one(speaker_id.T, speaker_id_table, b_dim, l_dim)
    out_p = _embed_one(phone.T, phone_table, b_dim, l_dim)
    return (out_s.transpose(1, 0, 2), out_p.transpose(1, 0, 2))
